# baseline (speedup 1.0000x reference)
import functools

import jax
import jax.numpy as jnp
from jax import lax
from jax.experimental import pallas as pl
from jax.experimental.pallas import tpu as pltpu
from jax.experimental.pallas import tpu_sc as plsc

D = 64
LANES = 16
VPR = D // LANES
NC, NS = 2, 16
NW = NC * NS
CHUNK = 1024
SUB = 128
NSUB = CHUNK // SUB
EPS = 1e-5


def _rsqrt(x):
    i = lax.bitcast_convert_type(x, jnp.int32)
    y = lax.bitcast_convert_type(jnp.int32(0x5F3759DF) - (i >> 1), jnp.float32)
    for _ in range(3):
        y = y * (1.5 - 0.5 * x * y * y)
    return y


def _make_sc_kernel(n_rows):
    per_w = n_rows // NW
    n_chunks = per_w // CHUNK
    mesh = plsc.VectorSubcoreMesh(core_axis_name="c", subcore_axis_name="s")

    @functools.partial(
        pl.kernel,
        mesh=mesh,
        out_type=jax.ShapeDtypeStruct((n_rows, D), jnp.float32),
        compiler_params=pltpu.CompilerParams(use_tc_tiling_on_sc=False),
        scratch_types=[
            pltpu.VMEM((NSUB, SUB), jnp.int32),
            pltpu.VMEM((CHUNK, D), jnp.float32),
            pltpu.VMEM((2, D), jnp.float32),
            pltpu.SemaphoreType.DMA,
        ],
    )
    def sc_kernel(idx_hbm, table_hbm, gamma_hbm, beta_hbm, out_hbm,
                  idx_v, rows_v, gb_v, sem):
        w = lax.axis_index("s") * NC + lax.axis_index("c")
        pltpu.sync_copy(gamma_hbm, gb_v.at[0])
        pltpu.sync_copy(beta_hbm, gb_v.at[1])
        g = [gb_v[0, pl.ds(j * LANES, LANES)] for j in range(VPR)]
        b = [gb_v[1, pl.ds(j * LANES, LANES)] for j in range(VPR)]
        base = w * per_w

        def do_chunk(c, carry):
            off = base + c * CHUNK
            pltpu.sync_copy(
                idx_hbm.at[pl.ds(pl.multiple_of(off // SUB, 8), NSUB)], idx_v)
            copies = [
                pltpu.async_copy(
                    table_hbm.at[idx_v.at[s]],
                    rows_v.at[pl.ds(s * SUB, SUB)],
                    sem,
                )
                for s in range(NSUB)
            ]
            for cp in copies:
                cp.wait()

            lanes = lax.iota(jnp.int32, LANES)
            dnums = lax.GatherDimensionNumbers(
                offset_dims=(), collapsed_slice_dims=(0,),
                start_index_map=(0,))

            def permute(x, idx):
                return lax.gather(
                    x, idx[:, None], dnums, (1,),
                    mode=lax.GatherScatterMode.PROMISE_IN_BOUNDS)

            def hsum(x):
                for k in (8, 4, 2, 1):
                    x = x + permute(x, lanes ^ k)
                return x

            def do_row(r, rc):
                v = [rows_v[r, pl.ds(j * LANES, LANES)] for j in range(VPR)]
                s0 = (v[0] + v[1]) + (v[2] + v[3])
                mean = hsum(s0) * (1.0 / D)
                d = [vj - mean for vj in v]
                sq = (d[0] * d[0] + d[1] * d[1]) + (d[2] * d[2] + d[3] * d[3])
                var = hsum(sq) * (1.0 / D)
                scale = _rsqrt(var + EPS)
                for j in range(VPR):
                    rows_v[r, pl.ds(j * LANES, LANES)] = d[j] * scale * g[j] + b[j]
                return rc

            lax.fori_loop(0, CHUNK, do_row, 0)
            pltpu.sync_copy(rows_v, out_hbm.at[pl.ds(off, CHUNK)])
            return carry

        lax.fori_loop(0, n_chunks, do_chunk, 0)

    return sc_kernel


@jax.jit
def kernel(idx, table, gamma, beta):
    B, L = idx.shape
    n_rows = B * L
    idx2d = idx.reshape(n_rows // SUB, SUB).astype(jnp.int32)
    out = _make_sc_kernel(n_rows)(idx2d, table, gamma, beta)
    return out.reshape(B, L, D)


# speedup vs baseline: 1.6382x; 1.6382x over previous
import functools

import jax
import jax.numpy as jnp
from jax import lax
from jax.experimental import pallas as pl
from jax.experimental.pallas import tpu as pltpu
from jax.experimental.pallas import tpu_sc as plsc

D = 64
LANES = 16
VPR = D // LANES
NC, NS = 2, 16
NW = NC * NS
CHUNK = 1024
SUB = 128
NSUB = CHUNK // SUB
EPS = 1e-5


def _rsqrt(x):
    i = lax.bitcast_convert_type(x, jnp.int32)
    y = lax.bitcast_convert_type(jnp.int32(0x5F3759DF) - (i >> 1), jnp.float32)
    for _ in range(3):
        y = y * (1.5 - 0.5 * x * y * y)
    return y


def _make_sc_kernel(n_rows):
    per_w = n_rows // NW
    n_chunks = per_w // CHUNK
    mesh = plsc.VectorSubcoreMesh(core_axis_name="c", subcore_axis_name="s")

    @functools.partial(
        pl.kernel,
        mesh=mesh,
        out_type=jax.ShapeDtypeStruct((n_rows, D), jnp.float32),
        compiler_params=pltpu.CompilerParams(use_tc_tiling_on_sc=False),
        scratch_types=[
            pltpu.VMEM((NSUB, SUB), jnp.int32),
            pltpu.VMEM((CHUNK, D), jnp.float32),
            pltpu.VMEM((2, D), jnp.float32),
            pltpu.SemaphoreType.DMA,
        ],
    )
    def sc_kernel(idx_hbm, table_hbm, gamma_hbm, beta_hbm, out_hbm,
                  idx_v, rows_v, gb_v, sem):
        w = lax.axis_index("s") * NC + lax.axis_index("c")
        pltpu.sync_copy(gamma_hbm, gb_v.at[0])
        pltpu.sync_copy(beta_hbm, gb_v.at[1])
        g = [gb_v[0, pl.ds(j * LANES, LANES)] for j in range(VPR)]
        b = [gb_v[1, pl.ds(j * LANES, LANES)] for j in range(VPR)]
        base = w * per_w

        def do_chunk(c, carry):
            off = base + c * CHUNK
            pltpu.sync_copy(
                idx_hbm.at[pl.ds(pl.multiple_of(off // SUB, 8), NSUB)], idx_v)
            copies = [
                pltpu.async_copy(
                    table_hbm.at[idx_v.at[s]],
                    rows_v.at[pl.ds(s * SUB, SUB)],
                    sem,
                )
                for s in range(NSUB)
            ]
            for cp in copies:
                cp.wait()

            lanes = lax.iota(jnp.int32, LANES)
            dnums = lax.GatherDimensionNumbers(
                offset_dims=(), collapsed_slice_dims=(0,),
                start_index_map=(0,))

            def permute(x, idx):
                return lax.gather(
                    x, idx[:, None], dnums, (1,),
                    mode=lax.GatherScatterMode.PROMISE_IN_BOUNDS)

            def hsum(x):
                for k in (8, 4, 2, 1):
                    x = x + permute(x, lanes ^ k)
                return x

            def do_row(r, rc):
                v = [rows_v[r, pl.ds(j * LANES, LANES)] for j in range(VPR)]
                s0 = (v[0] + v[1]) + (v[2] + v[3])
                sq = (v[0] * v[0] + v[1] * v[1]) + (v[2] * v[2] + v[3] * v[3])
                mean = hsum(s0) * (1.0 / D)
                msq = hsum(sq) * (1.0 / D)
                var = msq - mean * mean
                scale = _rsqrt(var + EPS)
                for j in range(VPR):
                    rows_v[r, pl.ds(j * LANES, LANES)] = (
                        (v[j] - mean) * (scale * g[j]) + b[j])
                return rc

            lax.fori_loop(0, CHUNK, do_row, 0, unroll=4)
            pltpu.sync_copy(rows_v, out_hbm.at[pl.ds(off, CHUNK)])
            return carry

        lax.fori_loop(0, n_chunks, do_chunk, 0)

    return sc_kernel


@jax.jit
def kernel(idx, table, gamma, beta):
    B, L = idx.shape
    n_rows = B * L
    idx2d = idx.reshape(n_rows // SUB, SUB).astype(jnp.int32)
    out = _make_sc_kernel(n_rows)(idx2d, table, gamma, beta)
    return out.reshape(B, L, D)


# speedup vs baseline: 1.7441x; 1.0646x over previous
import functools

import jax
import jax.numpy as jnp
from jax import lax
from jax.experimental import pallas as pl
from jax.experimental.pallas import tpu as pltpu
from jax.experimental.pallas import tpu_sc as plsc

D = 64
LANES = 16
VPR = D // LANES
NC, NS = 2, 16
NW = NC * NS
CHUNK = 512
SUB = 128
NSUB = CHUNK // SUB
EPS = 1e-5


def _rsqrt(x):
    i = lax.bitcast_convert_type(x, jnp.int32)
    y = lax.bitcast_convert_type(jnp.int32(0x5F3759DF) - (i >> 1), jnp.float32)
    for _ in range(3):
        y = y * (1.5 - 0.5 * x * y * y)
    return y


def _make_sc_kernel(n_rows):
    per_w = n_rows // NW
    n_chunks = per_w // CHUNK
    idx_rows = per_w // SUB
    mesh = plsc.VectorSubcoreMesh(core_axis_name="c", subcore_axis_name="s")

    @functools.partial(
        pl.kernel,
        mesh=mesh,
        out_type=jax.ShapeDtypeStruct((n_rows, D), jnp.float32),
        compiler_params=pltpu.CompilerParams(use_tc_tiling_on_sc=False),
        scratch_types=[
            pltpu.VMEM((idx_rows, SUB), jnp.int32),
            pltpu.VMEM((2, CHUNK, D), jnp.float32),
            pltpu.VMEM((2, D), jnp.float32),
            pltpu.SemaphoreType.DMA,
            pltpu.SemaphoreType.DMA,
        ],
    )
    def sc_kernel(idx_hbm, table_hbm, gamma_hbm, beta_hbm, out_hbm,
                  idx_v, rows_v, gb_v, gsem, osem):
        w = lax.axis_index("s") * NC + lax.axis_index("c")
        pltpu.sync_copy(gamma_hbm, gb_v.at[0])
        pltpu.sync_copy(beta_hbm, gb_v.at[1])
        g = [gb_v[0, pl.ds(j * LANES, LANES)] for j in range(VPR)]
        b = [gb_v[1, pl.ds(j * LANES, LANES)] for j in range(VPR)]
        base = w * per_w
        pltpu.sync_copy(
            idx_hbm.at[pl.ds(pl.multiple_of(w * idx_rows, 8), idx_rows)],
            idx_v)

        def fire_gather(c, par):
            for s in range(NSUB):
                pltpu.async_copy(
                    table_hbm.at[idx_v.at[c * NSUB + s]],
                    rows_v.at[par, pl.ds(s * SUB, SUB)],
                    gsem,
                )

        lanes = lax.iota(jnp.int32, LANES)
        dnums = lax.GatherDimensionNumbers(
            offset_dims=(), collapsed_slice_dims=(0,), start_index_map=(0,))

        def permute(x, idx):
            return lax.gather(
                x, idx[:, None], dnums, (1,),
                mode=lax.GatherScatterMode.PROMISE_IN_BOUNDS)

        def hsum(x):
            for k in (8, 4, 2, 1):
                x = x + permute(x, lanes ^ k)
            return x

        fire_gather(0, 0)

        def do_chunk(c, carry):
            par = c & 1
            pltpu.make_async_copy(
                out_hbm.at[pl.ds(0, CHUNK)], rows_v.at[par], gsem).wait()

            @pl.when(c + 1 < n_chunks)
            def _():
                @pl.when(c > 0)
                def _():
                    pltpu.make_async_copy(
                        rows_v.at[1 - par],
                        out_hbm.at[pl.ds(0, CHUNK)], osem).wait()

                fire_gather(c + 1, 1 - par)

            def do_row(r, rc):
                v = [rows_v[par, r, pl.ds(j * LANES, LANES)]
                     for j in range(VPR)]
                s0 = (v[0] + v[1]) + (v[2] + v[3])
                sq = (v[0] * v[0] + v[1] * v[1]) + (v[2] * v[2] + v[3] * v[3])
                mean = hsum(s0) * (1.0 / D)
                msq = hsum(sq) * (1.0 / D)
                var = msq - mean * mean
                scale = _rsqrt(var + EPS)
                for j in range(VPR):
                    rows_v[par, r, pl.ds(j * LANES, LANES)] = (
                        (v[j] - mean) * (scale * g[j]) + b[j])
                return rc

            lax.fori_loop(0, CHUNK, do_row, 0, unroll=4)

            pltpu.async_copy(
                rows_v.at[par],
                out_hbm.at[pl.ds(base + c * CHUNK, CHUNK)], osem)
            return carry

        lax.fori_loop(0, n_chunks, do_chunk, 0)
        for p in range(2):
            pltpu.make_async_copy(
                rows_v.at[p],
                out_hbm.at[pl.ds(0, CHUNK)], osem).wait()

    return sc_kernel


@jax.jit
def kernel(idx, table, gamma, beta):
    B, L = idx.shape
    n_rows = B * L
    idx2d = idx.reshape(n_rows // SUB, SUB).astype(jnp.int32)
    out = _make_sc_kernel(n_rows)(idx2d, table, gamma, beta)
    return out.reshape(B, L, D)
